# baseline (speedup 1.0000x reference)
import jax
import jax.numpy as jnp
from jax.experimental import pallas as pl
from jax.experimental.pallas import tpu as pltpu

_K = 64
_W = 250
_MR = 125
_RW = 2
_NQ = 4


def _mono(v):
    u = jax.lax.bitcast_convert_type(v, jnp.int32)
    return u ^ (jax.lax.shift_right_arithmetic(u, 31) & jnp.int32(0x7FFFFFFF))


def _rows_kernel(x_ref, o_ref, cand_ref):
    _BIG = jnp.int32(2**30)
    _INT_MIN = jnp.int32(-(2**31))
    x = x_ref[...]
    RW, R, MR, W = x.shape
    QR = R // _NQ

    def cnt(pred):
        return jnp.sum(pred.astype(jnp.int32), axis=(1, 2), keepdims=True)

    ci = _mono(jnp.max(x, axis=3))

    zero3 = jnp.zeros((RW, 1, 1), jnp.int32)
    tc = jnp.where(cnt(ci >= 0) >= _K, zero3, zero3 + _INT_MIN)

    def tc_body(b, t):
        t_try = t + jax.lax.shift_left(jnp.int32(1), jnp.int32(30) - b)
        return jnp.where(cnt(ci >= t_try) >= _K, t_try, t)

    tc = jax.lax.fori_loop(0, 31, tc_body, tc)
    s_sel = cnt(ci >= tc)

    _OFF = jnp.int32(8192)
    enc = (jax.lax.broadcasted_iota(jnp.int32, (RW, R, MR), 1) * 128
           + jax.lax.broadcasted_iota(jnp.int32, (RW, R, MR), 2))
    mi_all = jnp.where(ci > tc, enc,
                       jnp.where(ci == tc, enc + _OFF, _BIG))
    row64 = jax.lax.broadcasted_iota(jnp.int32, (_K, 1), 0)
    cb_init = jnp.zeros((_K, 1), jnp.int32)

    chains = []
    for r in range(_RW):
        tcr = tc[r, 0, 0]
        for q in range(_NQ):
            ciq = ci[r, q * QR:(q + 1) * QR, :]
            g = jnp.sum((ciq > tcr).astype(jnp.int32))
            e = jnp.sum((ciq == tcr).astype(jnp.int32))
            chains.append((r, q, mi_all[r, q * QR:(q + 1) * QR, :], g, e))

    slot_info = []
    for r in range(_RW):
        gs = [chains[r * _NQ + q][3] for q in range(_NQ)]
        es = [chains[r * _NQ + q][4] for q in range(_NQ)]
        g_tot = gs[0] + gs[1] + gs[2] + gs[3]
        pg = jnp.int32(0)
        pe = jnp.int32(0)
        for q in range(_NQ):
            keep_e = jnp.clip(_K - g_tot - pe, 0, es[q])
            bg = pg
            be = g_tot + jnp.minimum(pe, _K - g_tot)
            slot_info.append((bg, be, gs[q] + keep_e))
            pg = pg + gs[q]
            pe = pe + es[q]

    trip = slot_info[0][2]
    for ch in range(1, _RW * _NQ):
        trip = jnp.maximum(trip, slot_info[ch][2])

    def g_cond(st):
        return st[0] < trip

    def g_body(st):
        j, mis, cb0, cb1 = st
        new_mis = []
        cbs = [cb0, cb1]
        for ch in range(_RW * _NQ):
            r, q, _, g, _ = chains[ch]
            bg, be, need = slot_info[ch]
            mi = mis[ch]
            pv = jnp.min(mi)
            pos = pv & jnp.int32(8191)
            i_idx = jnp.minimum(pos >> 7, jnp.int32(R - 1))
            j_idx = jnp.minimum(pos & jnp.int32(127), jnp.int32(MR - 1))
            slot = jnp.where(j < g, bg + j,
                             jnp.where(j < need, be + (j - g), jnp.int32(_K)))
            cand_ref[r, pl.ds(slot, 1), :] = (
                x_ref[r, pl.ds(i_idx, 1), pl.ds(j_idx, 1), :].reshape(1, W))
            cflat = i_idx * jnp.int32(MR) + j_idx
            cbs[r] = jnp.where(row64 == slot, cflat, cbs[r])
            new_mis.append(jnp.where(mi == pv, _BIG, mi))
        return j + 1, tuple(new_mis), cbs[0], cbs[1]

    mis0 = tuple(ch[2] for ch in chains)
    _, _, cb0, cb1 = jax.lax.while_loop(
        g_cond, g_body, (jnp.int32(0), mis0, cb_init, cb_init))

    candi = _mono(cand_ref[:, 0:_K, :])

    def t_body(b, t):
        t_try = t + jax.lax.shift_left(jnp.int32(1), jnp.int32(30) - b)
        return jnp.where(cnt(candi >= t_try) >= _K, t_try, t)

    tt = jnp.where(cnt(candi >= 0) >= _K, zero3, zero3 + _INT_MIN)
    tt = jax.lax.fori_loop(0, 31, t_body, tt)

    c_above = cnt(candi > tt)
    cnt_eq = cnt(candi == tt)
    e_keep = _K - c_above
    t_f = jax.lax.bitcast_convert_type(
        tt ^ (jax.lax.shift_right_arithmetic(tt, 31) & jnp.int32(0x7FFFFFFF)),
        jnp.float32)
    t_f4 = t_f.reshape(RW, 1, 1, 1)

    fast = jnp.logical_and(
        cnt_eq == e_keep,
        jnp.logical_or(tt > tc, s_sel == _K))
    fast_all = jnp.all(fast)

    @pl.when(fast_all)
    def _fast():
        o_ref[...] = (x >= t_f4).astype(jnp.float32)

    @pl.when(jnp.logical_not(fast_all))
    def _slow():
        lane = jax.lax.broadcasted_iota(jnp.int32, (RW, _K, W), 2)
        cbs = jnp.stack([cb0, cb1])
        flat = cbs * W + lane
        eq = candi == tt

        def i_body(b, lo):
            add = jax.lax.shift_left(jnp.int32(1), jnp.int32(19) - b)
            i_mid = lo + add - 1
            c = cnt(jnp.logical_and(eq, flat <= i_mid))
            return jnp.where(c >= e_keep, lo, lo + add)

        i_e = jax.lax.fori_loop(0, 20, i_body, zero3)
        i_e4 = i_e.reshape(RW, 1, 1, 1)
        full_iota = (
            (jax.lax.broadcasted_iota(jnp.int32, (RW, R, MR, W), 1) * MR
             + jax.lax.broadcasted_iota(jnp.int32, (RW, R, MR, W), 2)) * W
            + jax.lax.broadcasted_iota(jnp.int32, (RW, R, MR, W), 3))
        keep = jnp.logical_or(
            x > t_f4, jnp.logical_and(x == t_f4, full_iota <= i_e4))
        o_ref[...] = keep.astype(jnp.float32)


def kernel(logits):
    B, N = logits.shape
    C = N // _W
    R = C // _MR
    x4 = logits.reshape(B, R, _MR, _W)
    out = pl.pallas_call(
        _rows_kernel,
        grid=(B // _RW,),
        in_specs=[pl.BlockSpec((_RW, R, _MR, _W), lambda i: (i, 0, 0, 0))],
        out_specs=pl.BlockSpec((_RW, R, _MR, _W), lambda i: (i, 0, 0, 0)),
        out_shape=jax.ShapeDtypeStruct((B, R, _MR, _W), jnp.float32),
        scratch_shapes=[pltpu.VMEM((_RW, _K + 1, _W), jnp.float32)],
        compiler_params=pltpu.CompilerParams(
            dimension_semantics=("arbitrary",),
        ),
    )(x4)
    return out.reshape(B, N)
